# Initial kernel scaffold; baseline (speedup 1.0000x reference)
import jax
import jax.numpy as jnp
from jax.experimental import pallas as pl


def kernel(g, in_feat, W1, al1, ar1, b1, W2, al2, ar2, b2, lw1, lb1, lw2, lb2, lw3, lb3, lw4, lb4, lw5, lb5):
    raise NotImplementedError("write your pallas kernel here")


# speedup vs baseline: 17.8275x; 17.8275x over previous
import functools

import jax
import jax.numpy as jnp
from jax import lax
from jax.experimental import pallas as pl
from jax.experimental.pallas import tpu as pltpu
from jax.experimental.pallas import tpu_sc as plsc

N = 10000
E = 320000
H = 3
D = 128
NC = 2
NS = 16
ET = E // (NC * NS)
K = 80
NCHUNK = ET // K
ROWS_W = E // K
STRIPE = N // NS
F32 = jnp.float32
I32 = jnp.int32


_GRID = 10
_R = N // _GRID


def _leaky(x, slope):
    return jnp.where(x >= 0, x, slope * x)


def _prep_body(x_ref, w_ref, al_ref, ar_ref, f_ref, ta_ref, er_ref):
    x = x_ref[...]
    w = w_ref[...]
    al = al_ref[...]
    ar = ar_ref[...]
    els, ers = [], []
    for h in range(H):
        fh = jnp.dot(x, w[:, h * D:(h + 1) * D], preferred_element_type=F32)
        f_ref[h] = fh
        els.append(jnp.sum(fh * al[h][None, :], axis=1))
        ers.append(jnp.sum(fh * ar[h][None, :], axis=1))
    z = jnp.zeros((x.shape[0], 16 - H), F32)
    ta_ref[...] = jnp.concatenate([jnp.stack(els, axis=1), z], axis=1)
    er_ref[...] = jnp.concatenate([jnp.stack(ers, axis=1), z], axis=1)


def _tc_prep(x, w, al, ar):
    return pl.pallas_call(
        _prep_body,
        grid=(_GRID,),
        in_specs=[
            pl.BlockSpec((_R, x.shape[1]), lambda i: (i, 0)),
            pl.BlockSpec(w.shape, lambda i: (0, 0)),
            pl.BlockSpec(al.shape, lambda i: (0, 0)),
            pl.BlockSpec(ar.shape, lambda i: (0, 0)),
        ],
        out_specs=[
            pl.BlockSpec((H, _R, D), lambda i: (0, i, 0)),
            pl.BlockSpec((_R, 16), lambda i: (i, 0)),
            pl.BlockSpec((_R, 16), lambda i: (i, 0)),
        ],
        out_shape=[
            jax.ShapeDtypeStruct((H, N, D), F32),
            jax.ShapeDtypeStruct((N, 16), F32),
            jax.ShapeDtypeStruct((N, 16), F32),
        ],
    )(x, w, al, ar)


def _tb_body(ta_ref, er_ref, tb_ref):
    el = ta_ref[...]
    er = er_ref[...]
    m = jnp.max(el[:, 0:H], axis=0)
    mp = _leaky(er[:, 0:H] + m[None, :], 0.2)
    z = jnp.zeros((el.shape[0], 16 - H), F32)
    tb_ref[...] = jnp.concatenate([er[:, 0:H], z, mp, z], axis=1)


def _tc_tb(ta, er):
    return pl.pallas_call(
        _tb_body,
        out_shape=jax.ShapeDtypeStruct((N, 32), F32),
    )(ta, er)


def _combine(acc_ref, s_ref, b_ref):
    s = s_ref[0] + s_ref[1]
    inv = 1.0 / (s[:, 0:H] + 1e-9)
    b = b_ref[...]
    hm = jnp.zeros((s.shape[0], D), F32)
    for h in range(H):
        o = (acc_ref[0, h] + acc_ref[1, h]) * inv[:, h][:, None] + b[h][None, :]
        hm = hm + _leaky(o, 0.01)
    return hm * (1.0 / H)


def _post_body(acc_ref, s_ref, b_ref, w_ref, al_ref, ar_ref,
               f_ref, ta_ref, er_ref):
    hm = _combine(acc_ref, s_ref, b_ref)
    w = w_ref[...]
    al = al_ref[...]
    ar = ar_ref[...]
    els, ers = [], []
    for h in range(H):
        fh = jnp.dot(hm, w[:, h * D:(h + 1) * D], preferred_element_type=F32)
        f_ref[h] = fh
        els.append(jnp.sum(fh * al[h][None, :], axis=1))
        ers.append(jnp.sum(fh * ar[h][None, :], axis=1))
    z = jnp.zeros((hm.shape[0], 16 - H), F32)
    ta_ref[...] = jnp.concatenate([jnp.stack(els, axis=1), z], axis=1)
    er_ref[...] = jnp.concatenate([jnp.stack(ers, axis=1), z], axis=1)


def _tc_post(acc, s, b, w, al, ar):
    return pl.pallas_call(
        _post_body,
        grid=(_GRID,),
        in_specs=[
            pl.BlockSpec((NC, H, _R, D), lambda i: (0, 0, i, 0)),
            pl.BlockSpec((NC, _R, 16), lambda i: (0, i, 0)),
            pl.BlockSpec(b.shape, lambda i: (0, 0)),
            pl.BlockSpec(w.shape, lambda i: (0, 0)),
            pl.BlockSpec(al.shape, lambda i: (0, 0)),
            pl.BlockSpec(ar.shape, lambda i: (0, 0)),
        ],
        out_specs=[
            pl.BlockSpec((H, _R, D), lambda i: (0, i, 0)),
            pl.BlockSpec((_R, 16), lambda i: (i, 0)),
            pl.BlockSpec((_R, 16), lambda i: (i, 0)),
        ],
        out_shape=[
            jax.ShapeDtypeStruct((H, N, D), F32),
            jax.ShapeDtypeStruct((N, 16), F32),
            jax.ShapeDtypeStruct((N, 16), F32),
        ],
    )(acc, s, b, w, al, ar)


def _mlp_body(acc_ref, s_ref, b_ref, w1_ref, b1_ref, w2_ref, b2_ref,
              w3_ref, b3_ref, w4_ref, b4_ref, w5_ref, b5_ref, o_ref):
    hm = _combine(acc_ref, s_ref, b_ref)
    hx = _leaky(jnp.dot(hm, w1_ref[...], preferred_element_type=F32)
                + b1_ref[...][None, :], 0.01)
    hx = _leaky(jnp.dot(hx, w2_ref[...], preferred_element_type=F32)
                + b2_ref[...][None, :], 0.01)
    hx = _leaky(jnp.dot(hx, w3_ref[...], preferred_element_type=F32)
                + b3_ref[...][None, :], 0.01)
    hx = _leaky(jnp.dot(hx, w4_ref[...], preferred_element_type=F32)
                + b4_ref[...][None, :], 0.01)
    o_ref[...] = (jnp.dot(hx, w5_ref[...], preferred_element_type=F32)
                  + b5_ref[...][None, :])


def _tc_mlp(acc, s, b, lw1, lb1, lw2, lb2, lw3, lb3, lw4, lb4, lw5, lb5):
    c = lw5.shape[1]
    return pl.pallas_call(
        _mlp_body,
        grid=(_GRID,),
        in_specs=[
            pl.BlockSpec((NC, H, _R, D), lambda i: (0, 0, i, 0)),
            pl.BlockSpec((NC, _R, 16), lambda i: (0, i, 0)),
            pl.BlockSpec(b.shape, lambda i: (0, 0)),
            pl.BlockSpec(lw1.shape, lambda i: (0, 0)),
            pl.BlockSpec(lb1.shape, lambda i: (0,)),
            pl.BlockSpec(lw2.shape, lambda i: (0, 0)),
            pl.BlockSpec(lb2.shape, lambda i: (0,)),
            pl.BlockSpec(lw3.shape, lambda i: (0, 0)),
            pl.BlockSpec(lb3.shape, lambda i: (0,)),
            pl.BlockSpec(lw4.shape, lambda i: (0, 0)),
            pl.BlockSpec(lb4.shape, lambda i: (0,)),
            pl.BlockSpec(lw5.shape, lambda i: (0, 0)),
            pl.BlockSpec(lb5.shape, lambda i: (0,)),
        ],
        out_specs=pl.BlockSpec((_R, c), lambda i: (i, 0)),
        out_shape=jax.ShapeDtypeStruct((N, c), F32),
    )(acc, s, b, lw1, lb1, lw2, lb2, lw3, lb3, lw4, lb4, lw5, lb5)


_MESH = plsc.VectorSubcoreMesh(core_axis_name="c", subcore_axis_name="s")
_SC_PARAMS = pltpu.CompilerParams(use_tc_tiling_on_sc=False)


@functools.partial(
    pl.kernel,
    mesh=_MESH,
    compiler_params=_SC_PARAMS,
    out_type=[
        jax.ShapeDtypeStruct((E, 16), F32),
        jax.ShapeDtypeStruct((NC, NS, STRIPE, 16), F32),
    ],
    scratch_types=[
        pltpu.VMEM((NCHUNK, K), I32),
        pltpu.VMEM((NCHUNK, K), I32),
        pltpu.VMEM((K, 16), F32),
        pltpu.VMEM((K, 32), F32),
        pltpu.VMEM((K, 16), F32),
        pltpu.VMEM((STRIPE, 16), F32),
        pltpu.VMEM_SHARED((N, 16), F32),
        pltpu.SemaphoreType.DMA,
    ],
)
def _sc_pass1(ta_hbm, tb_hbm, srcg, dstg, a_out, s_out,
              srcv, dstv, rowsa, rowsb, srows, sbuf, s_acc, sem):
    cid = lax.axis_index("c")
    sid = lax.axis_index("s")
    j = cid * NS + sid
    pltpu.sync_copy(srcg.at[j], srcv)
    pltpu.sync_copy(dstg.at[j], dstv)

    def _zero16(i, ref):
        ref[i, :] = jnp.zeros((16,), F32)
        return ref

    lax.fori_loop(0, STRIPE, lambda i, _: (_zero16(i, sbuf), 0)[1], 0)
    pltpu.sync_copy(sbuf, s_acc.at[pl.ds(sid * STRIPE, STRIPE)])
    plsc.subcore_barrier()

    def chunk(c, _):
        pltpu.sync_copy(ta_hbm.at[srcv.at[c]], rowsa)
        pltpu.sync_copy(tb_hbm.at[dstv.at[c]], rowsb)

        def edge(r, _):
            va = rowsa[r]
            vb = rowsb[r, pl.ds(0, 16)]
            vm = rowsb[r, pl.ds(16, 16)]
            x = va + vb
            e = jnp.where(x >= 0, x, 0.2 * x)
            srows[r] = jnp.exp(e - vm)
            return 0

        lax.fori_loop(0, K, edge, 0, unroll=4)
        pltpu.sync_copy(srows, s_acc.at[dstv.at[c]], add=True)
        pltpu.sync_copy(srows, a_out.at[pl.ds(j * ET + c * K, K)])
        return 0

    lax.fori_loop(0, NCHUNK, chunk, 0)
    plsc.subcore_barrier()
    pltpu.sync_copy(s_acc.at[pl.ds(sid * STRIPE, STRIPE)], sbuf)
    pltpu.sync_copy(sbuf, s_out.at[cid, sid])


@functools.partial(
    pl.kernel,
    mesh=_MESH,
    compiler_params=_SC_PARAMS,
    out_type=jax.ShapeDtypeStruct((NC, H, NS * 5, STRIPE // 5, D), F32),
    scratch_types=[
        pltpu.VMEM((NCHUNK, K), I32),
        pltpu.VMEM((NCHUNK, K), I32),
        pltpu.VMEM((K, 16), F32),
        pltpu.VMEM((K, D), F32),
        pltpu.VMEM((STRIPE // 5, D), F32),
        pltpu.VMEM_SHARED((N, D), F32),
        pltpu.SemaphoreType.DMA,
    ],
)
def _sc_pass2(f0, f1, f2, srcg, dstg, a_hbm, acc_out,
              srcv, dstv, av, rows, fbuf, acc, sem):
    cid = lax.axis_index("c")
    sid = lax.axis_index("s")
    j = cid * NS + sid
    pltpu.sync_copy(srcg.at[j], srcv)
    pltpu.sync_copy(dstg.at[j], dstv)

    fb_rows = STRIPE // 5

    def zero_fbuf():
        def zrow(i, _):
            for q in range(D // 16):
                fbuf[i, pl.ds(q * 16, 16)] = jnp.zeros((16,), F32)
            return 0
        lax.fori_loop(0, fb_rows, zrow, 0)

    zero_fbuf()
    for h in range(H):
        fh = (f0, f1, f2)[h]
        for k in range(5):
            pltpu.sync_copy(fbuf, acc.at[pl.ds(sid * STRIPE + k * fb_rows,
                                               fb_rows)])
        plsc.subcore_barrier()

        def chunk(c, _):
            pltpu.sync_copy(fh.at[srcv.at[c]], rows)
            pltpu.sync_copy(a_hbm.at[pl.ds(j * ET + c * K, K)], av)

            def row(r, _):
                w = av[r][h]
                for q in range(D // 16):
                    rows[r, pl.ds(q * 16, 16)] = rows[r, pl.ds(q * 16, 16)] * w
                return 0

            lax.fori_loop(0, K, row, 0, unroll=4)
            pltpu.sync_copy(rows, acc.at[dstv.at[c]], add=True)
            return 0

        lax.fori_loop(0, NCHUNK, chunk, 0)
        plsc.subcore_barrier()
        for k in range(5):
            pltpu.sync_copy(acc.at[pl.ds(sid * STRIPE + k * fb_rows, fb_rows)],
                            fbuf)
            pltpu.sync_copy(fbuf, acc_out.at[cid, h, sid * 5 + k])
        if h < H - 1:
            zero_fbuf()
            plsc.subcore_barrier()


def kernel(g, in_feat, W1, al1, ar1, b1, W2, al2, ar2, b2,
           lw1, lb1, lw2, lb2, lw3, lb3, lw4, lb4, lw5, lb5):
    srcg = g[0].reshape(NC * NS, NCHUNK, K)
    dstg = g[1].reshape(NC * NS, NCHUNK, K)

    f1, ta1, er1 = _tc_prep(in_feat, W1, al1, ar1)
    tb1 = _tc_tb(ta1, er1)
    a1, s1 = _sc_pass1(ta1, tb1, srcg, dstg)
    s1 = s1.reshape(NC, N, 16)
    acc1 = _sc_pass2(f1[0], f1[1], f1[2], srcg, dstg, a1)
    acc1 = acc1.reshape(NC, H, N, D)

    f2, ta2, er2 = _tc_post(acc1, s1, b1, W2, al2, ar2)
    tb2 = _tc_tb(ta2, er2)
    a2, s2 = _sc_pass1(ta2, tb2, srcg, dstg)
    s2 = s2.reshape(NC, N, 16)
    acc2 = _sc_pass2(f2[0], f2[1], f2[2], srcg, dstg, a2)
    acc2 = acc2.reshape(NC, H, N, D)

    return _tc_mlp(acc2, s2, b2,
                   lw1, lb1, lw2, lb2, lw3, lb3, lw4, lb4, lw5, lb5)


# speedup vs baseline: 26.1939x; 1.4693x over previous
import functools

import jax
import jax.numpy as jnp
from jax import lax
from jax.experimental import pallas as pl
from jax.experimental.pallas import tpu as pltpu
from jax.experimental.pallas import tpu_sc as plsc

N = 10000
E = 320000
H = 3
D = 128
NC = 2
NS = 16
ET = E // (NC * NS)
K = 80
NCHUNK = ET // K
ROWS_W = E // K
STRIPE = N // NS
F32 = jnp.float32
I32 = jnp.int32


_GRID = 10
_R = N // _GRID


def _leaky(x, slope):
    return jnp.where(x >= 0, x, slope * x)


def _prep_body(x_ref, w_ref, al_ref, ar_ref, f_ref, ta_ref, er_ref):
    x = x_ref[...]
    w = w_ref[...]
    al = al_ref[...]
    ar = ar_ref[...]
    els, ers = [], []
    for h in range(H):
        fh = jnp.dot(x, w[:, h * D:(h + 1) * D], preferred_element_type=F32)
        f_ref[h] = fh
        els.append(jnp.sum(fh * al[h][None, :], axis=1))
        ers.append(jnp.sum(fh * ar[h][None, :], axis=1))
    z = jnp.zeros((x.shape[0], 16 - H), F32)
    ta_ref[...] = jnp.concatenate([jnp.stack(els, axis=1), z], axis=1)
    er_ref[...] = jnp.concatenate([jnp.stack(ers, axis=1), z], axis=1)


def _tc_prep(x, w, al, ar):
    return pl.pallas_call(
        _prep_body,
        grid=(_GRID,),
        in_specs=[
            pl.BlockSpec((_R, x.shape[1]), lambda i: (i, 0)),
            pl.BlockSpec(w.shape, lambda i: (0, 0)),
            pl.BlockSpec(al.shape, lambda i: (0, 0)),
            pl.BlockSpec(ar.shape, lambda i: (0, 0)),
        ],
        out_specs=[
            pl.BlockSpec((H, _R, D), lambda i: (0, i, 0)),
            pl.BlockSpec((_R, 16), lambda i: (i, 0)),
            pl.BlockSpec((_R, 16), lambda i: (i, 0)),
        ],
        out_shape=[
            jax.ShapeDtypeStruct((H, N, D), F32),
            jax.ShapeDtypeStruct((N, 16), F32),
            jax.ShapeDtypeStruct((N, 16), F32),
        ],
    )(x, w, al, ar)


def _tb_body(ta_ref, er_ref, tb_ref):
    el = ta_ref[...]
    er = er_ref[...]
    m = jnp.max(el[:, 0:H], axis=0)
    mp = _leaky(er[:, 0:H] + m[None, :], 0.2)
    z = jnp.zeros((el.shape[0], 16 - H), F32)
    tb_ref[...] = jnp.concatenate([er[:, 0:H], z, mp, z], axis=1)


def _tc_tb(ta, er):
    return pl.pallas_call(
        _tb_body,
        out_shape=jax.ShapeDtypeStruct((N, 32), F32),
    )(ta, er)


def _combine(acc_ref, s_ref, b_ref):
    s = s_ref[0] + s_ref[1]
    inv = 1.0 / (s[:, 0:H] + 1e-9)
    b = b_ref[...]
    hm = jnp.zeros((s.shape[0], D), F32)
    for h in range(H):
        o = (acc_ref[0, h] + acc_ref[1, h]) * inv[:, h][:, None] + b[h][None, :]
        hm = hm + _leaky(o, 0.01)
    return hm * (1.0 / H)


def _post_body(acc_ref, s_ref, b_ref, w_ref, al_ref, ar_ref,
               f_ref, ta_ref, er_ref):
    hm = _combine(acc_ref, s_ref, b_ref)
    w = w_ref[...]
    al = al_ref[...]
    ar = ar_ref[...]
    els, ers = [], []
    for h in range(H):
        fh = jnp.dot(hm, w[:, h * D:(h + 1) * D], preferred_element_type=F32)
        f_ref[h] = fh
        els.append(jnp.sum(fh * al[h][None, :], axis=1))
        ers.append(jnp.sum(fh * ar[h][None, :], axis=1))
    z = jnp.zeros((hm.shape[0], 16 - H), F32)
    ta_ref[...] = jnp.concatenate([jnp.stack(els, axis=1), z], axis=1)
    er_ref[...] = jnp.concatenate([jnp.stack(ers, axis=1), z], axis=1)


def _tc_post(acc, s, b, w, al, ar):
    return pl.pallas_call(
        _post_body,
        grid=(_GRID,),
        in_specs=[
            pl.BlockSpec((NC, H, _R, D), lambda i: (0, 0, i, 0)),
            pl.BlockSpec((NC, _R, 16), lambda i: (0, i, 0)),
            pl.BlockSpec(b.shape, lambda i: (0, 0)),
            pl.BlockSpec(w.shape, lambda i: (0, 0)),
            pl.BlockSpec(al.shape, lambda i: (0, 0)),
            pl.BlockSpec(ar.shape, lambda i: (0, 0)),
        ],
        out_specs=[
            pl.BlockSpec((H, _R, D), lambda i: (0, i, 0)),
            pl.BlockSpec((_R, 16), lambda i: (i, 0)),
            pl.BlockSpec((_R, 16), lambda i: (i, 0)),
        ],
        out_shape=[
            jax.ShapeDtypeStruct((H, N, D), F32),
            jax.ShapeDtypeStruct((N, 16), F32),
            jax.ShapeDtypeStruct((N, 16), F32),
        ],
    )(acc, s, b, w, al, ar)


def _mlp_body(acc_ref, s_ref, b_ref, w1_ref, b1_ref, w2_ref, b2_ref,
              w3_ref, b3_ref, w4_ref, b4_ref, w5_ref, b5_ref, o_ref):
    hm = _combine(acc_ref, s_ref, b_ref)
    hx = _leaky(jnp.dot(hm, w1_ref[...], preferred_element_type=F32)
                + b1_ref[...][None, :], 0.01)
    hx = _leaky(jnp.dot(hx, w2_ref[...], preferred_element_type=F32)
                + b2_ref[...][None, :], 0.01)
    hx = _leaky(jnp.dot(hx, w3_ref[...], preferred_element_type=F32)
                + b3_ref[...][None, :], 0.01)
    hx = _leaky(jnp.dot(hx, w4_ref[...], preferred_element_type=F32)
                + b4_ref[...][None, :], 0.01)
    o_ref[...] = (jnp.dot(hx, w5_ref[...], preferred_element_type=F32)
                  + b5_ref[...][None, :])


def _tc_mlp(acc, s, b, lw1, lb1, lw2, lb2, lw3, lb3, lw4, lb4, lw5, lb5):
    c = lw5.shape[1]
    return pl.pallas_call(
        _mlp_body,
        grid=(_GRID,),
        in_specs=[
            pl.BlockSpec((NC, H, _R, D), lambda i: (0, 0, i, 0)),
            pl.BlockSpec((NC, _R, 16), lambda i: (0, i, 0)),
            pl.BlockSpec(b.shape, lambda i: (0, 0)),
            pl.BlockSpec(lw1.shape, lambda i: (0, 0)),
            pl.BlockSpec(lb1.shape, lambda i: (0,)),
            pl.BlockSpec(lw2.shape, lambda i: (0, 0)),
            pl.BlockSpec(lb2.shape, lambda i: (0,)),
            pl.BlockSpec(lw3.shape, lambda i: (0, 0)),
            pl.BlockSpec(lb3.shape, lambda i: (0,)),
            pl.BlockSpec(lw4.shape, lambda i: (0, 0)),
            pl.BlockSpec(lb4.shape, lambda i: (0,)),
            pl.BlockSpec(lw5.shape, lambda i: (0, 0)),
            pl.BlockSpec(lb5.shape, lambda i: (0,)),
        ],
        out_specs=pl.BlockSpec((_R, c), lambda i: (i, 0)),
        out_shape=jax.ShapeDtypeStruct((N, c), F32),
    )(acc, s, b, lw1, lb1, lw2, lb2, lw3, lb3, lw4, lb4, lw5, lb5)


_MESH = plsc.VectorSubcoreMesh(core_axis_name="c", subcore_axis_name="s")
_SC_PARAMS = pltpu.CompilerParams(use_tc_tiling_on_sc=False)


@functools.partial(
    pl.kernel,
    mesh=_MESH,
    compiler_params=_SC_PARAMS,
    out_type=[
        jax.ShapeDtypeStruct((E, 16), F32),
        jax.ShapeDtypeStruct((NC, NS, STRIPE, 16), F32),
    ],
    scratch_types=[
        pltpu.VMEM((NCHUNK, K), I32),
        pltpu.VMEM((NCHUNK, K), I32),
        pltpu.VMEM((K, 16), F32),
        pltpu.VMEM((K, 32), F32),
        pltpu.VMEM((K, 16), F32),
        pltpu.VMEM((STRIPE, 16), F32),
        pltpu.VMEM_SHARED((N, 16), F32),
        pltpu.SemaphoreType.DMA,
    ],
)
def _sc_pass1(ta_hbm, tb_hbm, srcg, dstg, a_out, s_out,
              srcv, dstv, rowsa, rowsb, srows, sbuf, s_acc, sem):
    cid = lax.axis_index("c")
    sid = lax.axis_index("s")
    j = cid * NS + sid
    pltpu.sync_copy(srcg.at[j], srcv)
    pltpu.sync_copy(dstg.at[j], dstv)

    def _zero16(i, ref):
        ref[i, :] = jnp.zeros((16,), F32)
        return ref

    lax.fori_loop(0, STRIPE, lambda i, _: (_zero16(i, sbuf), 0)[1], 0)
    pltpu.sync_copy(sbuf, s_acc.at[pl.ds(sid * STRIPE, STRIPE)])
    plsc.subcore_barrier()

    def chunk(c, _):
        pltpu.sync_copy(ta_hbm.at[srcv.at[c]], rowsa)
        pltpu.sync_copy(tb_hbm.at[dstv.at[c]], rowsb)

        def edge(r, _):
            va = rowsa[r]
            vb = rowsb[r, pl.ds(0, 16)]
            vm = rowsb[r, pl.ds(16, 16)]
            x = va + vb
            e = jnp.where(x >= 0, x, 0.2 * x)
            srows[r] = jnp.exp(e - vm)
            return 0

        lax.fori_loop(0, K, edge, 0, unroll=4)
        pltpu.sync_copy(srows, s_acc.at[dstv.at[c]], add=True)
        pltpu.sync_copy(srows, a_out.at[pl.ds(j * ET + c * K, K)])
        return 0

    lax.fori_loop(0, NCHUNK, chunk, 0)
    plsc.subcore_barrier()
    pltpu.sync_copy(s_acc.at[pl.ds(sid * STRIPE, STRIPE)], sbuf)
    pltpu.sync_copy(sbuf, s_out.at[cid, sid])


@functools.partial(
    pl.kernel,
    mesh=_MESH,
    compiler_params=_SC_PARAMS,
    out_type=jax.ShapeDtypeStruct((NC, H, NS * 25, STRIPE // 25, D), F32),
    scratch_types=[
        pltpu.VMEM((NCHUNK, K), I32),
        pltpu.VMEM((NCHUNK, K), I32),
        pltpu.VMEM((K, 16), F32),
        pltpu.VMEM((K, 16), F32),
        pltpu.VMEM((K, D), F32),
        pltpu.VMEM((K, D), F32),
        pltpu.VMEM((STRIPE // 25, D), F32),
        pltpu.VMEM_SHARED((N, D), F32),
        pltpu.SemaphoreType.DMA,
        pltpu.SemaphoreType.DMA,
        pltpu.SemaphoreType.DMA,
        pltpu.SemaphoreType.DMA,
    ],
)
def _sc_pass2(f0, f1, f2, srcg, dstg, a_hbm, acc_out,
              srcv, dstv, av0, av1, rows0, rows1, fbuf, acc,
              gsem0, gsem1, ssem0, ssem1):
    cid = lax.axis_index("c")
    sid = lax.axis_index("s")
    j = cid * NS + sid
    pltpu.sync_copy(srcg.at[j], srcv)
    pltpu.sync_copy(dstg.at[j], dstv)

    fb_rows = STRIPE // 25
    rows_ = (rows0, rows1)
    av_ = (av0, av1)
    gs_ = (gsem0, gsem1)
    ss_ = (ssem0, ssem1)

    def zero_fbuf():
        def zrow(i, _):
            for q in range(D // 16):
                fbuf[i, pl.ds(q * 16, 16)] = jnp.zeros((16,), F32)
            return 0
        lax.fori_loop(0, fb_rows, zrow, 0)

    zero_fbuf()
    for h in range(H):
        fh = (f0, f1, f2)[h]
        for k in range(25):
            pltpu.sync_copy(fbuf, acc.at[pl.ds(sid * STRIPE + k * fb_rows,
                                               fb_rows)])
        plsc.subcore_barrier()

        def issue(c, p):
            pltpu.async_copy(fh.at[srcv.at[c]], rows_[p], gs_[p])
            pltpu.async_copy(a_hbm.at[pl.ds(j * ET + c * K, K)], av_[p],
                             gs_[p])

        def proc(c, p):
            pltpu.make_async_copy(fh.at[srcv.at[c]], rows_[p], gs_[p]).wait()
            pltpu.make_async_copy(a_hbm.at[pl.ds(j * ET + c * K, K)],
                                  av_[p], gs_[p]).wait()

            @pl.when(c + 1 < NCHUNK)
            def _():
                @pl.when(c >= 1)
                def _():
                    pltpu.make_async_copy(rows_[1 - p], acc.at[dstv.at[0]],
                                          ss_[1 - p]).wait()
                issue(c + 1, 1 - p)

            def row(r, _):
                w = av_[p][r][h]
                for q in range(D // 16):
                    rows_[p][r, pl.ds(q * 16, 16)] = (
                        rows_[p][r, pl.ds(q * 16, 16)] * w)
                return 0

            lax.fori_loop(0, K, row, 0, unroll=4)
            pltpu.async_copy(rows_[p], acc.at[dstv.at[c]], ss_[p], add=True)

        issue(0, 0)

        def pair(m, _):
            for p in range(2):
                c = 2 * m + p

                @pl.when(c < NCHUNK)
                def _():
                    proc(c, p)
            return 0

        lax.fori_loop(0, (NCHUNK + 1) // 2, pair, 0)
        pltpu.make_async_copy(rows_[0], acc.at[dstv.at[0]], ss_[0]).wait()
        pltpu.make_async_copy(rows_[1], acc.at[dstv.at[0]], ss_[1]).wait()
        plsc.subcore_barrier()
        for k in range(25):
            pltpu.sync_copy(acc.at[pl.ds(sid * STRIPE + k * fb_rows, fb_rows)],
                            fbuf)
            pltpu.sync_copy(fbuf, acc_out.at[cid, h, sid * 25 + k])
        if h < H - 1:
            zero_fbuf()
            plsc.subcore_barrier()


def kernel(g, in_feat, W1, al1, ar1, b1, W2, al2, ar2, b2,
           lw1, lb1, lw2, lb2, lw3, lb3, lw4, lb4, lw5, lb5):
    srcg = g[0].reshape(NC * NS, NCHUNK, K)
    dstg = g[1].reshape(NC * NS, NCHUNK, K)

    f1, ta1, er1 = _tc_prep(in_feat, W1, al1, ar1)
    tb1 = _tc_tb(ta1, er1)
    a1, s1 = _sc_pass1(ta1, tb1, srcg, dstg)
    s1 = s1.reshape(NC, N, 16)
    acc1 = _sc_pass2(f1[0], f1[1], f1[2], srcg, dstg, a1)
    acc1 = acc1.reshape(NC, H, N, D)

    f2, ta2, er2 = _tc_post(acc1, s1, b1, W2, al2, ar2)
    tb2 = _tc_tb(ta2, er2)
    a2, s2 = _sc_pass1(ta2, tb2, srcg, dstg)
    s2 = s2.reshape(NC, N, 16)
    acc2 = _sc_pass2(f2[0], f2[1], f2[2], srcg, dstg, a2)
    acc2 = acc2.reshape(NC, H, N, D)

    return _tc_mlp(acc2, s2, b2,
                   lw1, lb1, lw2, lb2, lw3, lb3, lw4, lb4, lw5, lb5)


# speedup vs baseline: 31.3450x; 1.1967x over previous
import functools

import jax
import jax.numpy as jnp
from jax import lax
from jax.experimental import pallas as pl
from jax.experimental.pallas import tpu as pltpu
from jax.experimental.pallas import tpu_sc as plsc

N = 10000
E = 320000
H = 3
D = 128
NC = 2
NS = 16
ET = E // (NC * NS)
K = 80
NCHUNK = ET // K
ROWS_W = E // K
STRIPE = N // NS
F32 = jnp.float32
I32 = jnp.int32


_GRID = 10
_R = N // _GRID


def _leaky(x, slope):
    return jnp.where(x >= 0, x, slope * x)


def _prep_body(x_ref, w_ref, al_ref, ar_ref, f_ref, ta_ref, er_ref):
    x = x_ref[...]
    w = w_ref[...]
    al = al_ref[...]
    ar = ar_ref[...]
    els, ers = [], []
    for h in range(H):
        fh = jnp.dot(x, w[:, h * D:(h + 1) * D], preferred_element_type=F32)
        f_ref[h] = fh
        els.append(jnp.sum(fh * al[h][None, :], axis=1))
        ers.append(jnp.sum(fh * ar[h][None, :], axis=1))
    z = jnp.zeros((x.shape[0], 16 - H), F32)
    ta_ref[...] = jnp.concatenate([jnp.stack(els, axis=1), z], axis=1)
    er_ref[...] = jnp.concatenate([jnp.stack(ers, axis=1), z], axis=1)


def _tc_prep(x, w, al, ar):
    return pl.pallas_call(
        _prep_body,
        grid=(_GRID,),
        in_specs=[
            pl.BlockSpec((_R, x.shape[1]), lambda i: (i, 0)),
            pl.BlockSpec(w.shape, lambda i: (0, 0)),
            pl.BlockSpec(al.shape, lambda i: (0, 0)),
            pl.BlockSpec(ar.shape, lambda i: (0, 0)),
        ],
        out_specs=[
            pl.BlockSpec((H, _R, D), lambda i: (0, i, 0)),
            pl.BlockSpec((_R, 16), lambda i: (i, 0)),
            pl.BlockSpec((_R, 16), lambda i: (i, 0)),
        ],
        out_shape=[
            jax.ShapeDtypeStruct((H, N, D), F32),
            jax.ShapeDtypeStruct((N, 16), F32),
            jax.ShapeDtypeStruct((N, 16), F32),
        ],
    )(x, w, al, ar)


def _tb_body(ta_ref, er_ref, tb_ref):
    el = ta_ref[...]
    er = er_ref[...]
    m = jnp.max(el[:, 0:H], axis=0)
    mp = _leaky(er[:, 0:H] + m[None, :], 0.2)
    z = jnp.zeros((el.shape[0], 16 - H), F32)
    tb_ref[...] = jnp.concatenate([er[:, 0:H], z, mp, z], axis=1)


def _tc_tb(ta, er):
    return pl.pallas_call(
        _tb_body,
        out_shape=jax.ShapeDtypeStruct((N, 32), F32),
    )(ta, er)


def _combine(acc_ref, s_ref, b_ref):
    s = s_ref[0] + s_ref[1]
    inv = 1.0 / (s[:, 0:H] + 1e-9)
    b = b_ref[...]
    hm = jnp.zeros((s.shape[0], D), F32)
    for h in range(H):
        o = (acc_ref[0, h] + acc_ref[1, h]) * inv[:, h][:, None] + b[h][None, :]
        hm = hm + _leaky(o, 0.01)
    return hm * (1.0 / H)


def _post_body(acc_ref, s_ref, b_ref, w_ref, al_ref, ar_ref,
               f_ref, ta_ref, er_ref):
    hm = _combine(acc_ref, s_ref, b_ref)
    w = w_ref[...]
    al = al_ref[...]
    ar = ar_ref[...]
    els, ers = [], []
    for h in range(H):
        fh = jnp.dot(hm, w[:, h * D:(h + 1) * D], preferred_element_type=F32)
        f_ref[h] = fh
        els.append(jnp.sum(fh * al[h][None, :], axis=1))
        ers.append(jnp.sum(fh * ar[h][None, :], axis=1))
    z = jnp.zeros((hm.shape[0], 16 - H), F32)
    ta_ref[...] = jnp.concatenate([jnp.stack(els, axis=1), z], axis=1)
    er_ref[...] = jnp.concatenate([jnp.stack(ers, axis=1), z], axis=1)


def _tc_post(acc, s, b, w, al, ar):
    return pl.pallas_call(
        _post_body,
        grid=(_GRID,),
        in_specs=[
            pl.BlockSpec((NC, H, _R, D), lambda i: (0, 0, i, 0)),
            pl.BlockSpec((NC, _R, 16), lambda i: (0, i, 0)),
            pl.BlockSpec(b.shape, lambda i: (0, 0)),
            pl.BlockSpec(w.shape, lambda i: (0, 0)),
            pl.BlockSpec(al.shape, lambda i: (0, 0)),
            pl.BlockSpec(ar.shape, lambda i: (0, 0)),
        ],
        out_specs=[
            pl.BlockSpec((H, _R, D), lambda i: (0, i, 0)),
            pl.BlockSpec((_R, 16), lambda i: (i, 0)),
            pl.BlockSpec((_R, 16), lambda i: (i, 0)),
        ],
        out_shape=[
            jax.ShapeDtypeStruct((H, N, D), F32),
            jax.ShapeDtypeStruct((N, 16), F32),
            jax.ShapeDtypeStruct((N, 16), F32),
        ],
    )(acc, s, b, w, al, ar)


def _mlp_body(acc_ref, s_ref, b_ref, w1_ref, b1_ref, w2_ref, b2_ref,
              w3_ref, b3_ref, w4_ref, b4_ref, w5_ref, b5_ref, o_ref):
    hm = _combine(acc_ref, s_ref, b_ref)
    hx = _leaky(jnp.dot(hm, w1_ref[...], preferred_element_type=F32)
                + b1_ref[...][None, :], 0.01)
    hx = _leaky(jnp.dot(hx, w2_ref[...], preferred_element_type=F32)
                + b2_ref[...][None, :], 0.01)
    hx = _leaky(jnp.dot(hx, w3_ref[...], preferred_element_type=F32)
                + b3_ref[...][None, :], 0.01)
    hx = _leaky(jnp.dot(hx, w4_ref[...], preferred_element_type=F32)
                + b4_ref[...][None, :], 0.01)
    o_ref[...] = (jnp.dot(hx, w5_ref[...], preferred_element_type=F32)
                  + b5_ref[...][None, :])


def _tc_mlp(acc, s, b, lw1, lb1, lw2, lb2, lw3, lb3, lw4, lb4, lw5, lb5):
    c = lw5.shape[1]
    return pl.pallas_call(
        _mlp_body,
        grid=(_GRID,),
        in_specs=[
            pl.BlockSpec((NC, H, _R, D), lambda i: (0, 0, i, 0)),
            pl.BlockSpec((NC, _R, 16), lambda i: (0, i, 0)),
            pl.BlockSpec(b.shape, lambda i: (0, 0)),
            pl.BlockSpec(lw1.shape, lambda i: (0, 0)),
            pl.BlockSpec(lb1.shape, lambda i: (0,)),
            pl.BlockSpec(lw2.shape, lambda i: (0, 0)),
            pl.BlockSpec(lb2.shape, lambda i: (0,)),
            pl.BlockSpec(lw3.shape, lambda i: (0, 0)),
            pl.BlockSpec(lb3.shape, lambda i: (0,)),
            pl.BlockSpec(lw4.shape, lambda i: (0, 0)),
            pl.BlockSpec(lb4.shape, lambda i: (0,)),
            pl.BlockSpec(lw5.shape, lambda i: (0, 0)),
            pl.BlockSpec(lb5.shape, lambda i: (0,)),
        ],
        out_specs=pl.BlockSpec((_R, c), lambda i: (i, 0)),
        out_shape=jax.ShapeDtypeStruct((N, c), F32),
    )(acc, s, b, lw1, lb1, lw2, lb2, lw3, lb3, lw4, lb4, lw5, lb5)


_MESH = plsc.VectorSubcoreMesh(core_axis_name="c", subcore_axis_name="s")
_SC_PARAMS = pltpu.CompilerParams(use_tc_tiling_on_sc=False)


@functools.partial(
    pl.kernel,
    mesh=_MESH,
    compiler_params=_SC_PARAMS,
    out_type=[
        jax.ShapeDtypeStruct((E, 16), F32),
        jax.ShapeDtypeStruct((NC, NS, STRIPE, 16), F32),
    ],
    scratch_types=[
        pltpu.VMEM((NCHUNK, K), I32),
        pltpu.VMEM((NCHUNK, K), I32),
        pltpu.VMEM((K, 16), F32),
        pltpu.VMEM((K, 16), F32),
        pltpu.VMEM((K, 32), F32),
        pltpu.VMEM((K, 32), F32),
        pltpu.VMEM((K, 16), F32),
        pltpu.VMEM((STRIPE, 16), F32),
        pltpu.VMEM_SHARED((N, 16), F32),
        pltpu.SemaphoreType.DMA,
        pltpu.SemaphoreType.DMA,
    ],
)
def _sc_pass1(ta_hbm, tb_hbm, srcg, dstg, a_out, s_out,
              srcv, dstv, rowsa0, rowsa1, rowsb0, rowsb1, srows,
              sbuf, s_acc, gsem0, gsem1):
    cid = lax.axis_index("c")
    sid = lax.axis_index("s")
    j = cid * NS + sid
    pltpu.sync_copy(srcg.at[j], srcv)
    pltpu.sync_copy(dstg.at[j], dstv)

    def _zero16(i, ref):
        ref[i, :] = jnp.zeros((16,), F32)
        return ref

    lax.fori_loop(0, STRIPE, lambda i, _: (_zero16(i, sbuf), 0)[1], 0)
    pltpu.sync_copy(sbuf, s_acc.at[pl.ds(sid * STRIPE, STRIPE)])
    plsc.subcore_barrier()

    ra_ = (rowsa0, rowsa1)
    rb_ = (rowsb0, rowsb1)
    gs_ = (gsem0, gsem1)

    def issue(c, p):
        pltpu.async_copy(ta_hbm.at[srcv.at[c]], ra_[p], gs_[p])
        pltpu.async_copy(tb_hbm.at[dstv.at[c]], rb_[p], gs_[p])

    def proc(c, p):
        pltpu.make_async_copy(ta_hbm.at[srcv.at[c]], ra_[p], gs_[p]).wait()
        pltpu.make_async_copy(tb_hbm.at[dstv.at[c]], rb_[p], gs_[p]).wait()

        @pl.when(c + 1 < NCHUNK)
        def _():
            issue(c + 1, 1 - p)

        def edge(r, _):
            va = ra_[p][r]
            vb = rb_[p][r, pl.ds(0, 16)]
            vm = rb_[p][r, pl.ds(16, 16)]
            x = va + vb
            e = jnp.where(x >= 0, x, 0.2 * x)
            srows[r] = jnp.exp(e - vm)
            return 0

        lax.fori_loop(0, K, edge, 0, unroll=4)
        pltpu.sync_copy(srows, s_acc.at[dstv.at[c]], add=True)
        pltpu.sync_copy(srows, a_out.at[pl.ds(j * ET + c * K, K)])

    issue(0, 0)

    def pair(m, _):
        for p in range(2):
            c = 2 * m + p

            @pl.when(c < NCHUNK)
            def _():
                proc(c, p)
        return 0

    lax.fori_loop(0, (NCHUNK + 1) // 2, pair, 0)
    plsc.subcore_barrier()
    pltpu.sync_copy(s_acc.at[pl.ds(sid * STRIPE, STRIPE)], sbuf)
    pltpu.sync_copy(sbuf, s_out.at[cid, sid])


@functools.partial(
    pl.kernel,
    mesh=_MESH,
    compiler_params=_SC_PARAMS,
    out_type=jax.ShapeDtypeStruct((NC, H, NS * 25, STRIPE // 25, D), F32),
    scratch_types=[
        pltpu.VMEM((NCHUNK, K), I32),
        pltpu.VMEM((NCHUNK, K), I32),
        pltpu.VMEM((K, 16), F32),
        pltpu.VMEM((K, 16), F32),
        pltpu.VMEM((K, D), F32),
        pltpu.VMEM((K, D), F32),
        pltpu.VMEM((STRIPE // 25, D), F32),
        pltpu.VMEM_SHARED((N, D), F32),
        pltpu.SemaphoreType.DMA,
        pltpu.SemaphoreType.DMA,
        pltpu.SemaphoreType.DMA,
        pltpu.SemaphoreType.DMA,
    ],
)
def _sc_pass2(f0, f1, f2, srcg, dstg, a_hbm, acc_out,
              srcv, dstv, av0, av1, rows0, rows1, fbuf, acc,
              gsem0, gsem1, ssem0, ssem1):
    cid = lax.axis_index("c")
    sid = lax.axis_index("s")
    j = cid * NS + sid
    pltpu.sync_copy(srcg.at[j], srcv)
    pltpu.sync_copy(dstg.at[j], dstv)

    fb_rows = STRIPE // 25
    rows_ = (rows0, rows1)
    av_ = (av0, av1)
    gs_ = (gsem0, gsem1)
    ss_ = (ssem0, ssem1)

    def zero_fbuf():
        def zrow(i, _):
            for q in range(D // 16):
                fbuf[i, pl.ds(q * 16, 16)] = jnp.zeros((16,), F32)
            return 0
        lax.fori_loop(0, fb_rows, zrow, 0)

    zero_fbuf()
    for h in range(H):
        fh = (f0, f1, f2)[h]
        for k in range(25):
            pltpu.sync_copy(fbuf, acc.at[pl.ds(sid * STRIPE + k * fb_rows,
                                               fb_rows)])
        plsc.subcore_barrier()

        def issue(c, p):
            pltpu.async_copy(fh.at[srcv.at[c]], rows_[p], gs_[p])
            pltpu.async_copy(a_hbm.at[pl.ds(j * ET + c * K, K)], av_[p],
                             gs_[p])

        def proc(c, p):
            pltpu.make_async_copy(fh.at[srcv.at[c]], rows_[p], gs_[p]).wait()
            pltpu.make_async_copy(a_hbm.at[pl.ds(j * ET + c * K, K)],
                                  av_[p], gs_[p]).wait()

            @pl.when(c + 1 < NCHUNK)
            def _():
                @pl.when(c >= 1)
                def _():
                    pltpu.make_async_copy(rows_[1 - p], acc.at[dstv.at[0]],
                                          ss_[1 - p]).wait()
                issue(c + 1, 1 - p)

            def row(r, _):
                w = av_[p][r][h]
                for q in range(D // 16):
                    rows_[p][r, pl.ds(q * 16, 16)] = (
                        rows_[p][r, pl.ds(q * 16, 16)] * w)
                return 0

            lax.fori_loop(0, K, row, 0, unroll=4)
            pltpu.async_copy(rows_[p], acc.at[dstv.at[c]], ss_[p], add=True)

        issue(0, 0)

        def pair(m, _):
            for p in range(2):
                c = 2 * m + p

                @pl.when(c < NCHUNK)
                def _():
                    proc(c, p)
            return 0

        lax.fori_loop(0, (NCHUNK + 1) // 2, pair, 0)
        pltpu.make_async_copy(rows_[0], acc.at[dstv.at[0]], ss_[0]).wait()
        pltpu.make_async_copy(rows_[1], acc.at[dstv.at[0]], ss_[1]).wait()
        plsc.subcore_barrier()
        for k in range(25):
            pltpu.sync_copy(acc.at[pl.ds(sid * STRIPE + k * fb_rows, fb_rows)],
                            fbuf)
            pltpu.sync_copy(fbuf, acc_out.at[cid, h, sid * 25 + k])
        if h < H - 1:
            zero_fbuf()
            plsc.subcore_barrier()


def kernel(g, in_feat, W1, al1, ar1, b1, W2, al2, ar2, b2,
           lw1, lb1, lw2, lb2, lw3, lb3, lw4, lb4, lw5, lb5):
    srcg = g[0].reshape(NC * NS, NCHUNK, K)
    dstg = g[1].reshape(NC * NS, NCHUNK, K)

    f1, ta1, er1 = _tc_prep(in_feat, W1, al1, ar1)
    tb1 = _tc_tb(ta1, er1)
    a1, s1 = _sc_pass1(ta1, tb1, srcg, dstg)
    s1 = s1.reshape(NC, N, 16)
    acc1 = _sc_pass2(f1[0], f1[1], f1[2], srcg, dstg, a1)
    acc1 = acc1.reshape(NC, H, N, D)

    f2, ta2, er2 = _tc_post(acc1, s1, b1, W2, al2, ar2)
    tb2 = _tc_tb(ta2, er2)
    a2, s2 = _sc_pass1(ta2, tb2, srcg, dstg)
    s2 = s2.reshape(NC, N, 16)
    acc2 = _sc_pass2(f2[0], f2[1], f2[2], srcg, dstg, a2)
    acc2 = acc2.reshape(NC, H, N, D)

    return _tc_mlp(acc2, s2, b2,
                   lw1, lb1, lw2, lb2, lw3, lb3, lw4, lb4, lw5, lb5)
